# Initial kernel scaffold; baseline (speedup 1.0000x reference)
import jax
import jax.numpy as jnp
from jax.experimental import pallas as pl


def kernel(prefix, table):
    raise NotImplementedError("write your pallas kernel here")


# speedup vs baseline: 5.6058x; 5.6058x over previous
import jax
import jax.numpy as jnp
from jax import lax
from jax.experimental import pallas as pl
from jax.experimental.pallas import tpu as pltpu

_COL_CHUNK = 2048


def _onehot_gather_body(idx_ref, tab_ref, out_ref):
    n, _ = idx_ref.shape
    v = tab_ref.shape[0]
    iota = lax.broadcasted_iota(jnp.int32, (n, v), 1)
    onehot = (idx_ref[...] == iota).astype(jnp.bfloat16)
    tab = tab_ref[...].astype(jnp.bfloat16)
    acc = jnp.dot(onehot, tab, preferred_element_type=jnp.float32)
    out_ref[...] = acc.astype(jnp.bfloat16)


def kernel(prefix, table):
    b, p = prefix.shape
    v, d = table.shape
    n = b * p
    w = _COL_CHUNK
    grid = (d // w,)
    idx = prefix.reshape(n, 1).astype(jnp.int32)
    out = pl.pallas_call(
        _onehot_gather_body,
        grid=grid,
        in_specs=[
            pl.BlockSpec((n, 1), lambda j: (0, 0)),
            pl.BlockSpec((v, w), lambda j: (0, j)),
        ],
        out_specs=pl.BlockSpec((n, w), lambda j: (0, j)),
        out_shape=jax.ShapeDtypeStruct((n, d), jnp.bfloat16),
    )(idx, table)
    return out.reshape(b, p, d)
